# baseline (speedup 1.0000x reference)
import jax
import jax.numpy as jnp
from jax.experimental import pallas as pl
from jax.experimental.pallas import tpu as pltpu

_CHUNK = 256
_NBUF = 32
_AHEAD = 32


def _bounce_kernel(table_ref, out_ref, buf_ref, read_sems, write_sems):
    nch = out_ref.shape[0] // _CHUNK

    def read_copy(i):
        return pltpu.make_async_copy(
            table_ref.at[pl.ds(i * _CHUNK, _CHUNK)],
            buf_ref.at[i % _NBUF],
            read_sems.at[i % _NBUF],
        )

    def write_copy(i):
        return pltpu.make_async_copy(
            buf_ref.at[i % _NBUF],
            out_ref.at[pl.ds(i * _CHUNK, _CHUNK)],
            write_sems.at[i % _NBUF],
        )

    for i in range(min(_AHEAD, nch)):
        read_copy(i).start()
    for i in range(nch):
        read_copy(i).wait()
        write_copy(i).start()
        nxt = i + _AHEAD
        if nxt < nch:
            if nxt - _NBUF >= 0:
                write_copy(nxt - _NBUF).wait()
            read_copy(nxt).start()
    for i in range(max(nch - _NBUF, 0), nch):
        write_copy(i).wait()


def kernel(inputs, pos_table):
    seq_len = inputs.shape[-1]
    _, embed_dim = pos_table.shape
    return pl.pallas_call(
        _bounce_kernel,
        in_specs=[pl.BlockSpec(memory_space=pltpu.MemorySpace.HBM)],
        out_specs=pl.BlockSpec(memory_space=pltpu.MemorySpace.HBM),
        scratch_shapes=[
            pltpu.VMEM((_NBUF, _CHUNK, embed_dim), pos_table.dtype),
            pltpu.SemaphoreType.DMA((_NBUF,)),
            pltpu.SemaphoreType.DMA((_NBUF,)),
        ],
        out_shape=jax.ShapeDtypeStruct((seq_len, embed_dim), pos_table.dtype),
    )(pos_table)


# speedup vs baseline: 1.0204x; 1.0204x over previous
import jax
import jax.numpy as jnp
from jax.experimental import pallas as pl
from jax.experimental.pallas import tpu as pltpu

_CHUNK = 1024
_NBUF = 8
_AHEAD = 4


def _bounce_kernel(table_ref, out_ref, buf_ref, read_sems, write_sems):
    nch = out_ref.shape[0] // _CHUNK

    def read_copy(i):
        return pltpu.make_async_copy(
            table_ref.at[pl.ds(i * _CHUNK, _CHUNK)],
            buf_ref.at[i % _NBUF],
            read_sems.at[i % _NBUF],
        )

    def write_copy(i):
        return pltpu.make_async_copy(
            buf_ref.at[i % _NBUF],
            out_ref.at[pl.ds(i * _CHUNK, _CHUNK)],
            write_sems.at[i % _NBUF],
        )

    for i in range(min(_AHEAD, nch)):
        read_copy(i).start()
    for i in range(nch):
        read_copy(i).wait()
        write_copy(i).start()
        nxt = i + _AHEAD
        if nxt < nch:
            if nxt - _NBUF >= 0:
                write_copy(nxt - _NBUF).wait()
            read_copy(nxt).start()
    for i in range(max(nch - _NBUF, 0), nch):
        write_copy(i).wait()


def kernel(inputs, pos_table):
    seq_len = inputs.shape[-1]
    _, embed_dim = pos_table.shape
    return pl.pallas_call(
        _bounce_kernel,
        in_specs=[pl.BlockSpec(memory_space=pltpu.MemorySpace.HBM)],
        out_specs=pl.BlockSpec(memory_space=pltpu.MemorySpace.HBM),
        scratch_shapes=[
            pltpu.VMEM((_NBUF, _CHUNK, embed_dim), pos_table.dtype),
            pltpu.SemaphoreType.DMA((_NBUF,)),
            pltpu.SemaphoreType.DMA((_NBUF,)),
        ],
        out_shape=jax.ShapeDtypeStruct((seq_len, embed_dim), pos_table.dtype),
    )(pos_table)


# speedup vs baseline: 1.0251x; 1.0046x over previous
import jax
import jax.numpy as jnp
from jax.experimental import pallas as pl
from jax.experimental.pallas import tpu as pltpu

_CHUNK = 2048
_NBUF = 4
_AHEAD = 2


def _bounce_kernel(table_ref, out_ref, buf_ref, read_sems, write_sems):
    nch = out_ref.shape[0] // _CHUNK

    def read_copy(i):
        return pltpu.make_async_copy(
            table_ref.at[pl.ds(i * _CHUNK, _CHUNK)],
            buf_ref.at[i % _NBUF],
            read_sems.at[i % _NBUF],
        )

    def write_copy(i):
        return pltpu.make_async_copy(
            buf_ref.at[i % _NBUF],
            out_ref.at[pl.ds(i * _CHUNK, _CHUNK)],
            write_sems.at[i % _NBUF],
        )

    for i in range(min(_AHEAD, nch)):
        read_copy(i).start()
    for i in range(nch):
        read_copy(i).wait()
        write_copy(i).start()
        nxt = i + _AHEAD
        if nxt < nch:
            if nxt - _NBUF >= 0:
                write_copy(nxt - _NBUF).wait()
            read_copy(nxt).start()
    for i in range(max(nch - _NBUF, 0), nch):
        write_copy(i).wait()


def kernel(inputs, pos_table):
    seq_len = inputs.shape[-1]
    _, embed_dim = pos_table.shape
    return pl.pallas_call(
        _bounce_kernel,
        in_specs=[pl.BlockSpec(memory_space=pltpu.MemorySpace.HBM)],
        out_specs=pl.BlockSpec(memory_space=pltpu.MemorySpace.HBM),
        scratch_shapes=[
            pltpu.VMEM((_NBUF, _CHUNK, embed_dim), pos_table.dtype),
            pltpu.SemaphoreType.DMA((_NBUF,)),
            pltpu.SemaphoreType.DMA((_NBUF,)),
        ],
        out_shape=jax.ShapeDtypeStruct((seq_len, embed_dim), pos_table.dtype),
    )(pos_table)


# speedup vs baseline: 1.0480x; 1.0224x over previous
import jax
import jax.numpy as jnp
from jax.experimental import pallas as pl
from jax.experimental.pallas import tpu as pltpu

_CHUNK = 4096
_NBUF = 4
_AHEAD = 2


def _bounce_kernel(table_ref, out_ref, buf_ref, read_sems, write_sems):
    nch = out_ref.shape[0] // _CHUNK

    def read_copy(i):
        return pltpu.make_async_copy(
            table_ref.at[pl.ds(i * _CHUNK, _CHUNK)],
            buf_ref.at[i % _NBUF],
            read_sems.at[i % _NBUF],
        )

    def write_copy(i):
        return pltpu.make_async_copy(
            buf_ref.at[i % _NBUF],
            out_ref.at[pl.ds(i * _CHUNK, _CHUNK)],
            write_sems.at[i % _NBUF],
        )

    for i in range(min(_AHEAD, nch)):
        read_copy(i).start()
    for i in range(nch):
        read_copy(i).wait()
        write_copy(i).start()
        nxt = i + _AHEAD
        if nxt < nch:
            if nxt - _NBUF >= 0:
                write_copy(nxt - _NBUF).wait()
            read_copy(nxt).start()
    for i in range(max(nch - _NBUF, 0), nch):
        write_copy(i).wait()


def kernel(inputs, pos_table):
    seq_len = inputs.shape[-1]
    _, embed_dim = pos_table.shape
    return pl.pallas_call(
        _bounce_kernel,
        in_specs=[pl.BlockSpec(memory_space=pltpu.MemorySpace.HBM)],
        out_specs=pl.BlockSpec(memory_space=pltpu.MemorySpace.HBM),
        scratch_shapes=[
            pltpu.VMEM((_NBUF, _CHUNK, embed_dim), pos_table.dtype),
            pltpu.SemaphoreType.DMA((_NBUF,)),
            pltpu.SemaphoreType.DMA((_NBUF,)),
        ],
        out_shape=jax.ShapeDtypeStruct((seq_len, embed_dim), pos_table.dtype),
    )(pos_table)
